# baseline (speedup 1.0000x reference)
import jax
import jax.numpy as jnp
from jax import lax
from jax.experimental import pallas as pl
from jax.experimental.pallas import tpu as pltpu
from jax.experimental.pallas import tpu_sc as plsc

N = 10000
D = 128
E = 320000

NC = 2
NS = 16
NW = NC * NS
K = 64
EPAD = 327680
EPT = EPAD // NW
NCHUNK = EPT // K

NPAD = 10240
RPT = NPAD // NS
TRASH = N

_MESH = plsc.VectorSubcoreMesh(core_axis_name="c", subcore_axis_name="s")
_SC_PARAMS = pltpu.CompilerParams(needs_layout_passes=False)


def _deg_body(dst_hbm, deg_out, dst_v, acc_v, tbuf, rbuf, deg_sh, sem):
    c = lax.axis_index("c")
    s = lax.axis_index("s")
    wid = c * NS + s

    pltpu.async_copy(dst_hbm.at[wid], dst_v, sem).wait()

    z16 = jnp.zeros((16,), jnp.float32)

    @pl.loop(0, NPAD, step=16)
    def _(i):
        acc_v[pl.ds(i, 16)] = z16

    ones16 = jnp.ones((16,), jnp.float32)

    @pl.loop(0, EPT, step=16)
    def _(e):
        idx = dst_v[pl.ds(e, 16)]
        plsc.addupdate_scatter(acc_v, [idx], ones16)

    pltpu.sync_copy(acc_v, deg_sh.at[s])
    plsc.subcore_barrier()

    @pl.loop(0, RPT, step=16)
    def _(i):
        rbuf[pl.ds(i, 16)] = z16

    for j in range(NS):
        pltpu.sync_copy(deg_sh.at[j, pl.ds(s * RPT, RPT)], tbuf)

        @pl.loop(0, RPT, step=16)
        def _(i):
            rbuf[pl.ds(i, 16)] = rbuf[pl.ds(i, 16)] + tbuf[pl.ds(i, 16)]

    pltpu.sync_copy(rbuf, deg_out.at[c, pl.ds(s * RPT, RPT)])


_deg_kernel = pl.kernel(
    _deg_body,
    out_type=jax.ShapeDtypeStruct((NC, NPAD), jnp.float32),
    mesh=_MESH,
    compiler_params=_SC_PARAMS,
    scratch_types=[
        pltpu.VMEM((EPT,), jnp.int32),
        pltpu.VMEM((NPAD,), jnp.float32),
        pltpu.VMEM((RPT,), jnp.float32),
        pltpu.VMEM((RPT,), jnp.float32),
        pltpu.VMEM_SHARED((NS, NPAD), jnp.float32),
        pltpu.SemaphoreType.DMA,
    ],
)


def _agg_body(y_hbm, src_hbm, dst_hbm, out_hbm,
              src_v, dst_v, rbuf0, rbuf1, acc_sh, sem0, sem1, sem2, sem3):
    c = lax.axis_index("c")
    s = lax.axis_index("s")
    wid = c * NS + s

    pltpu.async_copy(src_hbm.at[wid], src_v, sem0).wait()
    pltpu.async_copy(dst_hbm.at[wid], dst_v, sem1).wait()

    z16 = jnp.zeros((16,), jnp.float32)

    @pl.loop(0, K)
    def _(r):
        for i in range(D // 16):
            rbuf0[r, pl.ds(i * 16, 16)] = z16

    for t in range(RPT // K):
        pltpu.sync_copy(rbuf0, acc_sh.at[pl.ds(s * RPT + t * K, K)])
    plsc.subcore_barrier()

    def gather(j, rbuf, sem):
        pltpu.async_copy(y_hbm.at[src_v.at[pl.ds(j * K, K)]], rbuf, sem)

    def gather_wait(j, rbuf, sem):
        pltpu.make_async_copy(y_hbm.at[src_v.at[pl.ds(j * K, K)]], rbuf,
                              sem).wait()

    def scatter_fire(j, rbuf, sem):
        for k in range(K // 16):
            idx = dst_v[pl.ds(j * K + k * 16, 16)]
            pltpu.async_copy(rbuf.at[pl.ds(k * 16, 16)], acc_sh.at[idx],
                             sem, add=True)

    def scatter_drain(j, rbuf, sem):
        for k in range(K // 16):
            idx = dst_v[pl.ds(j * K + k * 16, 16)]
            pltpu.make_async_copy(rbuf.at[pl.ds(k * 16, 16)],
                                  acc_sh.at[idx], sem).wait()

    gather(0, rbuf0, sem0)
    gather(1, rbuf1, sem1)

    @pl.loop(0, NCHUNK, step=2)
    def _(j):
        gather_wait(j, rbuf0, sem0)
        scatter_fire(j, rbuf0, sem2)
        gather_wait(j + 1, rbuf1, sem1)
        scatter_fire(j + 1, rbuf1, sem3)

        @pl.when(j + 2 < NCHUNK)
        def _():
            scatter_drain(j, rbuf0, sem2)
            gather(j + 2, rbuf0, sem0)
            scatter_drain(j + 1, rbuf1, sem3)
            gather(j + 3, rbuf1, sem1)

    scatter_drain(NCHUNK - 2, rbuf0, sem2)
    scatter_drain(NCHUNK - 1, rbuf1, sem3)
    plsc.subcore_barrier()

    for t in range(RPT // K):
        pltpu.sync_copy(acc_sh.at[pl.ds(s * RPT + t * K, K)], rbuf0)
        pltpu.sync_copy(rbuf0, out_hbm.at[c, pl.ds(s * RPT + t * K, K)])


_agg_kernel = pl.kernel(
    _agg_body,
    out_type=jax.ShapeDtypeStruct((NC, NPAD, D), jnp.float32),
    mesh=_MESH,
    compiler_params=_SC_PARAMS,
    scratch_types=[
        pltpu.VMEM((EPT,), jnp.int32),
        pltpu.VMEM((EPT,), jnp.int32),
        pltpu.VMEM((K, D), jnp.float32),
        pltpu.VMEM((K, D), jnp.float32),
        pltpu.VMEM_SHARED((NPAD, D), jnp.float32),
        pltpu.SemaphoreType.DMA,
        pltpu.SemaphoreType.DMA,
        pltpu.SemaphoreType.DMA,
        pltpu.SemaphoreType.DMA,
    ],
)


_GRID = NPAD // RPT


def _tc1_body(dega_ref, degb_ref, x_ref, w_ref, y_ref, d_ref):
    d = lax.rsqrt(1.0 + dega_ref[...] + degb_ref[...])
    xw = jnp.dot(x_ref[...], w_ref[...],
                 preferred_element_type=jnp.float32,
                 precision=lax.Precision.HIGHEST)
    y_ref[...] = xw * d
    d_ref[...] = d


def _tc2_body(a_ref, y1_ref, d_ref, b_ref, w_ref, y2_ref):
    d = d_ref[...]
    h = d * (a_ref[0] + a_ref[1] + y1_ref[...]) + b_ref[...]
    h = jnp.maximum(h, 0.0)
    y2_ref[...] = d * jnp.dot(h, w_ref[...],
                              preferred_element_type=jnp.float32,
                              precision=lax.Precision.HIGHEST)


def _tc3_body(a_ref, y2_ref, d_ref, b_ref, o_ref):
    o_ref[...] = d_ref[...] * (a_ref[0] + a_ref[1] + y2_ref[...]) + b_ref[...]


def _row_spec(shape_last):
    return pl.BlockSpec((RPT, shape_last), lambda i: (i, 0))


_AGG_SPEC = pl.BlockSpec((NC, RPT, D), lambda i: (0, i, 0))
_FULL_W = pl.BlockSpec((D, D), lambda i: (0, 0))
_FULL_B = pl.BlockSpec((1, D), lambda i: (0, 0))

_tc1 = pl.pallas_call(
    _tc1_body,
    grid=(_GRID,),
    in_specs=[_row_spec(1), _row_spec(1), _row_spec(D), _FULL_W],
    out_specs=[_row_spec(D), _row_spec(1)],
    out_shape=[jax.ShapeDtypeStruct((NPAD, D), jnp.float32),
               jax.ShapeDtypeStruct((NPAD, 1), jnp.float32)],
)

_tc2 = pl.pallas_call(
    _tc2_body,
    grid=(_GRID,),
    in_specs=[_AGG_SPEC, _row_spec(D), _row_spec(1), _FULL_B, _FULL_W],
    out_specs=_row_spec(D),
    out_shape=jax.ShapeDtypeStruct((NPAD, D), jnp.float32),
)

_tc3 = pl.pallas_call(
    _tc3_body,
    grid=(_GRID,),
    in_specs=[_AGG_SPEC, _row_spec(D), _row_spec(1), _FULL_B],
    out_specs=_row_spec(D),
    out_shape=jax.ShapeDtypeStruct((NPAD, D), jnp.float32),
)


@jax.jit
def kernel(x, edge_index, W1, b1, W2, b2):
    src = edge_index[0].astype(jnp.int32)
    dst = edge_index[1].astype(jnp.int32)
    src_p = jnp.concatenate(
        [src, jnp.zeros((EPAD - E,), jnp.int32)]).reshape(NW, EPT)
    dst_p = jnp.concatenate(
        [dst, jnp.full((EPAD - E,), TRASH, jnp.int32)]).reshape(NW, EPT)

    deg_p = _deg_kernel(dst_p)
    dega = deg_p[0].reshape(NPAD, 1)
    degb = deg_p[1].reshape(NPAD, 1)

    x_pad = jnp.pad(x, ((0, NPAD - N), (0, 0)))
    y1, dcol = _tc1(dega, degb, x_pad, W1)

    agg1 = _agg_kernel(y1, src_p, dst_p)
    y2 = _tc2(agg1, y1, dcol, b1.reshape(1, D), W2)

    agg2 = _agg_kernel(y2, src_p, dst_p)
    out = _tc3(agg2, y2, dcol, b2.reshape(1, D))
    return out[:N]


# speedup vs baseline: 2.5461x; 2.5461x over previous
import jax
import jax.numpy as jnp
from jax import lax
from jax.experimental import pallas as pl
from jax.experimental.pallas import tpu as pltpu
from jax.experimental.pallas import tpu_sc as plsc

N = 10000
D = 128
E = 320000

NC = 2
NS = 16
NW = NC * NS
K = 64
EPAD = 327680
EPT = EPAD // NW
NCHUNK = EPT // K

NPAD = 10240
RPT = NPAD // NS
TRASH = N

_MESH = plsc.VectorSubcoreMesh(core_axis_name="c", subcore_axis_name="s")
_SC_PARAMS = pltpu.CompilerParams(needs_layout_passes=False)


def _deg_body(dst_hbm, deg_out, dst_v, acc_v, tbuf, rbuf, deg_sh, sem):
    c = lax.axis_index("c")
    s = lax.axis_index("s")
    wid = c * NS + s

    pltpu.async_copy(dst_hbm.at[wid], dst_v, sem).wait()

    z16 = jnp.zeros((16,), jnp.float32)

    @pl.loop(0, NPAD, step=16)
    def _(i):
        acc_v[pl.ds(i, 16)] = z16

    ones16 = jnp.ones((16,), jnp.float32)

    @pl.loop(0, EPT, step=16)
    def _(e):
        idx = dst_v[pl.ds(e, 16)]
        plsc.addupdate_scatter(acc_v, [idx], ones16)

    pltpu.sync_copy(acc_v, deg_sh.at[s])
    plsc.subcore_barrier()

    @pl.loop(0, RPT, step=16)
    def _(i):
        rbuf[pl.ds(i, 16)] = z16

    for j in range(NS):
        pltpu.sync_copy(deg_sh.at[j, pl.ds(s * RPT, RPT)], tbuf)

        @pl.loop(0, RPT, step=16)
        def _(i):
            rbuf[pl.ds(i, 16)] = rbuf[pl.ds(i, 16)] + tbuf[pl.ds(i, 16)]

    pltpu.sync_copy(rbuf, deg_out.at[c, pl.ds(s * RPT, RPT)])


_deg_kernel = pl.kernel(
    _deg_body,
    out_type=jax.ShapeDtypeStruct((NC, NPAD), jnp.float32),
    mesh=_MESH,
    compiler_params=_SC_PARAMS,
    scratch_types=[
        pltpu.VMEM((EPT,), jnp.int32),
        pltpu.VMEM((NPAD,), jnp.float32),
        pltpu.VMEM((RPT,), jnp.float32),
        pltpu.VMEM((RPT,), jnp.float32),
        pltpu.VMEM_SHARED((NS, NPAD), jnp.float32),
        pltpu.SemaphoreType.DMA,
    ],
)


def _agg_body(y_hbm, src_hbm, dst_hbm, out_hbm,
              src_v, dst_v, rbuf0, rbuf1, acc_sh, sem0, sem1, sem2, sem3):
    c = lax.axis_index("c")
    s = lax.axis_index("s")
    wid = c * NS + s

    pltpu.async_copy(src_hbm.at[wid], src_v, sem0).wait()
    pltpu.async_copy(dst_hbm.at[wid], dst_v, sem1).wait()

    z16 = jnp.zeros((16,), jnp.float32)

    @pl.loop(0, K)
    def _(r):
        for i in range(D // 16):
            rbuf0[r, pl.ds(i * 16, 16)] = z16

    for t in range(RPT // K):
        pltpu.sync_copy(rbuf0, acc_sh.at[pl.ds(s * RPT + t * K, K)])
    plsc.subcore_barrier()

    def gather(j, rbuf, sem):
        pltpu.async_copy(y_hbm.at[src_v.at[pl.ds(j * K, K)]], rbuf, sem)

    def gather_wait(j, rbuf, sem):
        pltpu.make_async_copy(y_hbm.at[src_v.at[pl.ds(j * K, K)]], rbuf,
                              sem).wait()

    def scatter_fire(j, rbuf, sem):
        for k in range(K // 16):
            idx = dst_v[pl.ds(j * K + k * 16, 16)]
            pltpu.async_copy(rbuf.at[pl.ds(k * 16, 16)], acc_sh.at[idx],
                             sem, add=True)

    def scatter_drain(j, rbuf, sem):
        for k in range(K // 16):
            idx = dst_v[pl.ds(j * K + k * 16, 16)]
            pltpu.make_async_copy(rbuf.at[pl.ds(k * 16, 16)],
                                  acc_sh.at[idx], sem).wait()

    gather(0, rbuf0, sem0)
    gather(1, rbuf1, sem1)

    @pl.loop(0, NCHUNK, step=2)
    def _(j):
        gather_wait(j, rbuf0, sem0)
        scatter_fire(j, rbuf0, sem2)
        gather_wait(j + 1, rbuf1, sem1)
        scatter_fire(j + 1, rbuf1, sem3)

        @pl.when(j + 2 < NCHUNK)
        def _():
            scatter_drain(j, rbuf0, sem2)
            gather(j + 2, rbuf0, sem0)
            scatter_drain(j + 1, rbuf1, sem3)
            gather(j + 3, rbuf1, sem1)

    scatter_drain(NCHUNK - 2, rbuf0, sem2)
    scatter_drain(NCHUNK - 1, rbuf1, sem3)
    plsc.subcore_barrier()

    for t in range(RPT // K):
        pltpu.sync_copy(acc_sh.at[pl.ds(s * RPT + t * K, K)], rbuf0)
        pltpu.sync_copy(rbuf0, out_hbm.at[c, pl.ds(s * RPT + t * K, K)])


_agg_kernel = pl.kernel(
    _agg_body,
    out_type=jax.ShapeDtypeStruct((NC, NPAD, D), jnp.float32),
    mesh=_MESH,
    compiler_params=_SC_PARAMS,
    scratch_types=[
        pltpu.VMEM((EPT,), jnp.int32),
        pltpu.VMEM((EPT,), jnp.int32),
        pltpu.VMEM((K, D), jnp.float32),
        pltpu.VMEM((K, D), jnp.float32),
        pltpu.VMEM_SHARED((NPAD, D), jnp.float32),
        pltpu.SemaphoreType.DMA,
        pltpu.SemaphoreType.DMA,
        pltpu.SemaphoreType.DMA,
        pltpu.SemaphoreType.DMA,
    ],
)


_GRID = NPAD // RPT


def _tc1_body(dega_ref, degb_ref, x_ref, w_ref, y_ref, d_ref):
    d = lax.rsqrt(1.0 + dega_ref[...] + degb_ref[...])
    xw = jnp.dot(x_ref[...], w_ref[...],
                 preferred_element_type=jnp.float32,
                 precision=lax.Precision.HIGHEST)
    y_ref[...] = xw * d
    d_ref[...] = d


def _tc2_body(a_ref, y1_ref, d_ref, b_ref, w_ref, y2_ref):
    d = d_ref[...]
    h = d * (a_ref[0] + a_ref[1] + y1_ref[...]) + b_ref[...]
    h = jnp.maximum(h, 0.0)
    y2 = d * jnp.dot(h, w_ref[...],
                     preferred_element_type=jnp.float32,
                     precision=lax.Precision.HIGHEST)
    row = pl.program_id(0) * RPT + lax.broadcasted_iota(jnp.int32, (RPT, 1), 0)
    y2_ref[...] = jnp.where(row < N, y2, 0.0)


def _tc3_body(a_ref, y2_ref, d_ref, b_ref, o_ref):
    o_ref[...] = d_ref[...] * (a_ref[0] + a_ref[1] + y2_ref[...]) + b_ref[...]


def _row_spec(shape_last):
    return pl.BlockSpec((RPT, shape_last), lambda i: (i, 0))


_AGG_SPEC = pl.BlockSpec((NC, RPT, D), lambda i: (0, i, 0))
_FULL_W = pl.BlockSpec((D, D), lambda i: (0, 0))
_FULL_B = pl.BlockSpec((1, D), lambda i: (0, 0))

_tc1 = pl.pallas_call(
    _tc1_body,
    grid=(_GRID,),
    in_specs=[_row_spec(1), _row_spec(1), _row_spec(D), _FULL_W],
    out_specs=[_row_spec(D), _row_spec(1)],
    out_shape=[jax.ShapeDtypeStruct((NPAD, D), jnp.float32),
               jax.ShapeDtypeStruct((NPAD, 1), jnp.float32)],
)

_tc2 = pl.pallas_call(
    _tc2_body,
    grid=(_GRID,),
    in_specs=[_AGG_SPEC, _row_spec(D), _row_spec(1), _FULL_B, _FULL_W],
    out_specs=_row_spec(D),
    out_shape=jax.ShapeDtypeStruct((NPAD, D), jnp.float32),
)

_tc3 = pl.pallas_call(
    _tc3_body,
    grid=(_GRID,),
    in_specs=[_AGG_SPEC, _row_spec(D), _row_spec(1), _FULL_B],
    out_specs=_row_spec(D),
    out_shape=jax.ShapeDtypeStruct((NPAD, D), jnp.float32),
)


@jax.jit
def kernel(x, edge_index, W1, b1, W2, b2):
    src = edge_index[0].astype(jnp.int32)
    dst = edge_index[1].astype(jnp.int32)
    pad = jnp.arange(EPAD - E, dtype=jnp.int32)
    src_p = jnp.concatenate(
        [src, N + pad % (NPAD - N)]).reshape(NW, EPT)
    dst_p = jnp.concatenate(
        [dst, pad % N]).reshape(NW, EPT)
    dst_deg = jnp.concatenate(
        [dst, jnp.full((EPAD - E,), TRASH, jnp.int32)]).reshape(NW, EPT)

    deg_p = _deg_kernel(dst_deg)
    dega = deg_p[0].reshape(NPAD, 1)
    degb = deg_p[1].reshape(NPAD, 1)

    x_pad = jnp.pad(x, ((0, NPAD - N), (0, 0)))
    y1, dcol = _tc1(dega, degb, x_pad, W1)

    agg1 = _agg_kernel(y1, src_p, dst_p)
    y2 = _tc2(agg1, y1, dcol, b1.reshape(1, D), W2)

    agg2 = _agg_kernel(y2, src_p, dst_p)
    out = _tc3(agg2, y2, dcol, b2.reshape(1, D))
    return out[:N]


# speedup vs baseline: 2.7575x; 1.0830x over previous
import jax
import jax.numpy as jnp
from jax import lax
from jax.experimental import pallas as pl
from jax.experimental.pallas import tpu as pltpu
from jax.experimental.pallas import tpu_sc as plsc

N = 10000
D = 128
E = 320000

NC = 2
NS = 16
NW = NC * NS
K = 128
EPAD = 327680
EPT = EPAD // NW
NPHASE = 2
PCH = EPT // (NPHASE * K)
PE = PCH * K

NPAD = 10240
RPT = NPAD // NS
TRASH = N

_MESH = plsc.VectorSubcoreMesh(core_axis_name="c", subcore_axis_name="s")
_SC_PARAMS = pltpu.CompilerParams(needs_layout_passes=False)


def _deg_body(dst_hbm, deg_out, dst_v, acc_v, tbuf, rbuf, deg_sh, sem):
    c = lax.axis_index("c")
    s = lax.axis_index("s")
    wid = c * NS + s

    pltpu.async_copy(dst_hbm.at[wid], dst_v, sem).wait()

    z16 = jnp.zeros((16,), jnp.float32)

    @pl.loop(0, NPAD, step=16)
    def _(i):
        acc_v[pl.ds(i, 16)] = z16

    ones16 = jnp.ones((16,), jnp.float32)

    @pl.loop(0, EPT, step=16)
    def _(e):
        idx = dst_v[pl.ds(e, 16)]
        plsc.addupdate_scatter(acc_v, [idx], ones16)

    pltpu.sync_copy(acc_v, deg_sh.at[s])
    plsc.subcore_barrier()

    @pl.loop(0, RPT, step=16)
    def _(i):
        rbuf[pl.ds(i, 16)] = z16

    for j in range(NS):
        pltpu.sync_copy(deg_sh.at[j, pl.ds(s * RPT, RPT)], tbuf)

        @pl.loop(0, RPT, step=16)
        def _(i):
            rbuf[pl.ds(i, 16)] = rbuf[pl.ds(i, 16)] + tbuf[pl.ds(i, 16)]

    pltpu.sync_copy(rbuf, deg_out.at[c, pl.ds(s * RPT, RPT)])


_deg_kernel = pl.kernel(
    _deg_body,
    out_type=jax.ShapeDtypeStruct((NC, NPAD), jnp.float32),
    mesh=_MESH,
    compiler_params=_SC_PARAMS,
    scratch_types=[
        pltpu.VMEM((EPT,), jnp.int32),
        pltpu.VMEM((NPAD,), jnp.float32),
        pltpu.VMEM((RPT,), jnp.float32),
        pltpu.VMEM((RPT,), jnp.float32),
        pltpu.VMEM_SHARED((NS, NPAD), jnp.float32),
        pltpu.SemaphoreType.DMA,
    ],
)


def _agg_body(y_hbm, src_hbm, dst_hbm, out_hbm,
              src_v, dst_v, rbuf0, rbuf1, acc_sh, sem0, sem1, sem2, sem3):
    c = lax.axis_index("c")
    s = lax.axis_index("s")
    wid = c * NS + s

    z16 = jnp.zeros((16,), jnp.float32)

    @pl.loop(0, K)
    def _(r):
        for i in range(D // 16):
            rbuf0[r, pl.ds(i * 16, 16)] = z16

    for t in range(RPT // K):
        pltpu.sync_copy(rbuf0, acc_sh.at[pl.ds(s * RPT + t * K, K)])
    plsc.subcore_barrier()

    def gather(j, rbuf, sem):
        pltpu.async_copy(y_hbm.at[src_v.at[pl.ds(j * K, K)]], rbuf, sem)

    def gather_wait(j, rbuf, sem):
        pltpu.make_async_copy(y_hbm.at[src_v.at[pl.ds(j * K, K)]], rbuf,
                              sem).wait()

    def scatter_fire(j, rbuf, sem):
        pltpu.async_copy(rbuf, acc_sh.at[dst_v.at[j]], sem, add=True)

    def scatter_drain(j, rbuf, sem):
        pltpu.make_async_copy(rbuf, acc_sh.at[dst_v.at[j]], sem).wait()

    for p in range(NPHASE):
        pltpu.async_copy(src_hbm.at[wid, pl.ds(p * PE, PE)], src_v,
                         sem0).wait()
        pltpu.async_copy(dst_hbm.at[wid, p], dst_v, sem1).wait()
        gather(0, rbuf0, sem0)
        gather(1, rbuf1, sem1)

        @pl.loop(0, PCH, step=2)
        def _(j):
            gather_wait(j, rbuf0, sem0)
            scatter_fire(j, rbuf0, sem2)
            gather_wait(j + 1, rbuf1, sem1)
            scatter_fire(j + 1, rbuf1, sem3)

            @pl.when(j + 2 < PCH)
            def _():
                scatter_drain(j, rbuf0, sem2)
                gather(j + 2, rbuf0, sem0)
                scatter_drain(j + 1, rbuf1, sem3)
                gather(j + 3, rbuf1, sem1)

        scatter_drain(PCH - 2, rbuf0, sem2)
        scatter_drain(PCH - 1, rbuf1, sem3)

    plsc.subcore_barrier()

    for t in range(RPT // K):
        pltpu.sync_copy(acc_sh.at[pl.ds(s * RPT + t * K, K)], rbuf0)
        pltpu.sync_copy(rbuf0, out_hbm.at[c, pl.ds(s * RPT + t * K, K)])


_agg_kernel = pl.kernel(
    _agg_body,
    out_type=jax.ShapeDtypeStruct((NC, NPAD, D), jnp.float32),
    mesh=_MESH,
    compiler_params=_SC_PARAMS,
    scratch_types=[
        pltpu.VMEM((PE,), jnp.int32),
        pltpu.VMEM((PCH, K), jnp.int32),
        pltpu.VMEM((K, D), jnp.float32),
        pltpu.VMEM((K, D), jnp.float32),
        pltpu.VMEM_SHARED((NPAD, D), jnp.float32),
        pltpu.SemaphoreType.DMA,
        pltpu.SemaphoreType.DMA,
        pltpu.SemaphoreType.DMA,
        pltpu.SemaphoreType.DMA,
    ],
)


_GRID = NPAD // RPT


def _tc1_body(dega_ref, degb_ref, x_ref, w_ref, y_ref, d_ref):
    d = lax.rsqrt(1.0 + dega_ref[...] + degb_ref[...])
    xw = jnp.dot(x_ref[...], w_ref[...],
                 preferred_element_type=jnp.float32,
                 precision=lax.Precision.HIGHEST)
    y_ref[...] = xw * d
    d_ref[...] = d


def _tc2_body(a_ref, y1_ref, d_ref, b_ref, w_ref, y2_ref):
    d = d_ref[...]
    h = d * (a_ref[0] + a_ref[1] + y1_ref[...]) + b_ref[...]
    h = jnp.maximum(h, 0.0)
    y2 = d * jnp.dot(h, w_ref[...],
                     preferred_element_type=jnp.float32,
                     precision=lax.Precision.HIGHEST)
    row = pl.program_id(0) * RPT + lax.broadcasted_iota(jnp.int32, (RPT, 1), 0)
    y2_ref[...] = jnp.where(row < N, y2, 0.0)


def _tc3_body(a_ref, y2_ref, d_ref, b_ref, o_ref):
    o_ref[...] = d_ref[...] * (a_ref[0] + a_ref[1] + y2_ref[...]) + b_ref[...]


def _row_spec(shape_last):
    return pl.BlockSpec((RPT, shape_last), lambda i: (i, 0))


_AGG_SPEC = pl.BlockSpec((NC, RPT, D), lambda i: (0, i, 0))
_FULL_W = pl.BlockSpec((D, D), lambda i: (0, 0))
_FULL_B = pl.BlockSpec((1, D), lambda i: (0, 0))

_tc1 = pl.pallas_call(
    _tc1_body,
    grid=(_GRID,),
    in_specs=[_row_spec(1), _row_spec(1), _row_spec(D), _FULL_W],
    out_specs=[_row_spec(D), _row_spec(1)],
    out_shape=[jax.ShapeDtypeStruct((NPAD, D), jnp.float32),
               jax.ShapeDtypeStruct((NPAD, 1), jnp.float32)],
)

_tc2 = pl.pallas_call(
    _tc2_body,
    grid=(_GRID,),
    in_specs=[_AGG_SPEC, _row_spec(D), _row_spec(1), _FULL_B, _FULL_W],
    out_specs=_row_spec(D),
    out_shape=jax.ShapeDtypeStruct((NPAD, D), jnp.float32),
)

_tc3 = pl.pallas_call(
    _tc3_body,
    grid=(_GRID,),
    in_specs=[_AGG_SPEC, _row_spec(D), _row_spec(1), _FULL_B],
    out_specs=_row_spec(D),
    out_shape=jax.ShapeDtypeStruct((NPAD, D), jnp.float32),
)


@jax.jit
def kernel(x, edge_index, W1, b1, W2, b2):
    src = edge_index[0].astype(jnp.int32)
    dst = edge_index[1].astype(jnp.int32)
    pad = jnp.arange(EPAD - E, dtype=jnp.int32)
    src_p = jnp.concatenate(
        [src, N + pad % (NPAD - N)]).reshape(NW, EPT)
    dst_p = jnp.concatenate(
        [dst, pad % N]).reshape(NW, NPHASE, PCH, K)
    dst_deg = jnp.concatenate(
        [dst, jnp.full((EPAD - E,), TRASH, jnp.int32)]).reshape(NW, EPT)

    deg_p = _deg_kernel(dst_deg)
    dega = deg_p[0].reshape(NPAD, 1)
    degb = deg_p[1].reshape(NPAD, 1)

    x_pad = jnp.pad(x, ((0, NPAD - N), (0, 0)))
    y1, dcol = _tc1(dega, degb, x_pad, W1)

    agg1 = _agg_kernel(y1, src_p, dst_p)
    y2 = _tc2(agg1, y1, dcol, b1.reshape(1, D), W2)

    agg2 = _agg_kernel(y2, src_p, dst_p)
    out = _tc3(agg2, y2, dcol, b2.reshape(1, D))
    return out[:N]


# speedup vs baseline: 2.8010x; 1.0158x over previous
import jax
import jax.numpy as jnp
from jax import lax
from jax.experimental import pallas as pl
from jax.experimental.pallas import tpu as pltpu
from jax.experimental.pallas import tpu_sc as plsc

N = 10000
D = 128
E = 320000

NC = 2
NS = 16
NW = NC * NS
K = 128
EPAD = 327680
EPT = EPAD // NW
NPHASE = 2
PCH = EPT // (NPHASE * K)
PE = PCH * K

NPAD = 10240
RPT = NPAD // NS
TRASH = N

_MESH = plsc.VectorSubcoreMesh(core_axis_name="c", subcore_axis_name="s")
_SC_PARAMS = pltpu.CompilerParams(needs_layout_passes=False)


def _deg_body(dst_hbm, deg_out, dst_v, acc_v, tbuf, rbuf, deg_sh, sem):
    c = lax.axis_index("c")
    s = lax.axis_index("s")
    wid = c * NS + s

    pltpu.async_copy(dst_hbm.at[wid], dst_v, sem).wait()

    z16 = jnp.zeros((16,), jnp.float32)

    @pl.loop(0, NPAD, step=16)
    def _(i):
        acc_v[pl.ds(i, 16)] = z16

    ones16 = jnp.ones((16,), jnp.float32)

    @pl.loop(0, EPT, step=16)
    def _(e):
        idx = dst_v[pl.ds(e, 16)]
        plsc.addupdate_scatter(acc_v, [idx], ones16)

    pltpu.sync_copy(acc_v, deg_sh.at[s])
    plsc.subcore_barrier()

    @pl.loop(0, RPT, step=16)
    def _(i):
        rbuf[pl.ds(i, 16)] = z16

    for j in range(NS):
        pltpu.sync_copy(deg_sh.at[j, pl.ds(s * RPT, RPT)], tbuf)

        @pl.loop(0, RPT, step=16)
        def _(i):
            rbuf[pl.ds(i, 16)] = rbuf[pl.ds(i, 16)] + tbuf[pl.ds(i, 16)]

    pltpu.sync_copy(rbuf, deg_out.at[c, pl.ds(s * RPT, RPT)])


_deg_kernel = pl.kernel(
    _deg_body,
    out_type=jax.ShapeDtypeStruct((NC, NPAD), jnp.float32),
    mesh=_MESH,
    compiler_params=_SC_PARAMS,
    scratch_types=[
        pltpu.VMEM((EPT,), jnp.int32),
        pltpu.VMEM((NPAD,), jnp.float32),
        pltpu.VMEM((RPT,), jnp.float32),
        pltpu.VMEM((RPT,), jnp.float32),
        pltpu.VMEM_SHARED((NS, NPAD), jnp.float32),
        pltpu.SemaphoreType.DMA,
    ],
)


def _agg_body(y_hbm, src_hbm, dst_hbm, out_hbm,
              src_v, dst_v, rbuf0, rbuf1, acc_sh, sem0, sem1, sem2, sem3):
    c = lax.axis_index("c")
    s = lax.axis_index("s")
    wid = c * NS + s

    z16 = jnp.zeros((16,), jnp.float32)

    @pl.loop(0, K)
    def _(r):
        for i in range(D // 16):
            rbuf0[r, pl.ds(i * 16, 16)] = z16

    for t in range(RPT // K):
        pltpu.sync_copy(rbuf0, acc_sh.at[pl.ds(s * RPT + t * K, K)])
    plsc.subcore_barrier()

    H = K // 2

    def gather(j, rbuf, sem):
        pltpu.async_copy(y_hbm.at[src_v.at[pl.ds(j * K, H)]],
                         rbuf.at[pl.ds(0, H)], sem)
        pltpu.async_copy(y_hbm.at[src_v.at[pl.ds(j * K + H, H)]],
                         rbuf.at[pl.ds(H, H)], sem)

    def gather_wait(j, rbuf, sem):
        pltpu.make_async_copy(y_hbm.at[src_v.at[pl.ds(j * K, H)]],
                              rbuf.at[pl.ds(0, H)], sem).wait()
        pltpu.make_async_copy(y_hbm.at[src_v.at[pl.ds(j * K + H, H)]],
                              rbuf.at[pl.ds(H, H)], sem).wait()

    def scatter_fire(j, rbuf, sem):
        pltpu.async_copy(rbuf, acc_sh.at[dst_v.at[j]], sem, add=True)

    def scatter_drain(j, rbuf, sem):
        pltpu.make_async_copy(rbuf, acc_sh.at[dst_v.at[j]], sem).wait()

    for p in range(NPHASE):
        pltpu.async_copy(src_hbm.at[wid, pl.ds(p * PE, PE)], src_v,
                         sem0).wait()
        pltpu.async_copy(dst_hbm.at[wid, p], dst_v, sem1).wait()
        gather(0, rbuf0, sem0)
        gather(1, rbuf1, sem1)

        @pl.loop(0, PCH, step=2)
        def _(j):
            gather_wait(j, rbuf0, sem0)
            scatter_fire(j, rbuf0, sem2)
            gather_wait(j + 1, rbuf1, sem1)
            scatter_fire(j + 1, rbuf1, sem3)

            @pl.when(j + 2 < PCH)
            def _():
                scatter_drain(j, rbuf0, sem2)
                gather(j + 2, rbuf0, sem0)
                scatter_drain(j + 1, rbuf1, sem3)
                gather(j + 3, rbuf1, sem1)

        scatter_drain(PCH - 2, rbuf0, sem2)
        scatter_drain(PCH - 1, rbuf1, sem3)

    plsc.subcore_barrier()

    for t in range(RPT // K):
        pltpu.sync_copy(acc_sh.at[pl.ds(s * RPT + t * K, K)], rbuf0)
        pltpu.sync_copy(rbuf0, out_hbm.at[c, pl.ds(s * RPT + t * K, K)])


_agg_kernel = pl.kernel(
    _agg_body,
    out_type=jax.ShapeDtypeStruct((NC, NPAD, D), jnp.float32),
    mesh=_MESH,
    compiler_params=_SC_PARAMS,
    scratch_types=[
        pltpu.VMEM((PE,), jnp.int32),
        pltpu.VMEM((PCH, K), jnp.int32),
        pltpu.VMEM((K, D), jnp.float32),
        pltpu.VMEM((K, D), jnp.float32),
        pltpu.VMEM_SHARED((NPAD, D), jnp.float32),
        pltpu.SemaphoreType.DMA,
        pltpu.SemaphoreType.DMA,
        pltpu.SemaphoreType.DMA,
        pltpu.SemaphoreType.DMA,
    ],
)


_GRID = NPAD // RPT


def _tc1_body(dega_ref, degb_ref, x_ref, w_ref, y_ref, d_ref):
    d = lax.rsqrt(1.0 + dega_ref[...] + degb_ref[...])
    xw = jnp.dot(x_ref[...], w_ref[...],
                 preferred_element_type=jnp.float32,
                 precision=lax.Precision.HIGHEST)
    y_ref[...] = xw * d
    d_ref[...] = d


def _tc2_body(a_ref, y1_ref, d_ref, b_ref, w_ref, y2_ref):
    d = d_ref[...]
    h = d * (a_ref[0] + a_ref[1] + y1_ref[...]) + b_ref[...]
    h = jnp.maximum(h, 0.0)
    y2 = d * jnp.dot(h, w_ref[...],
                     preferred_element_type=jnp.float32,
                     precision=lax.Precision.HIGHEST)
    row = pl.program_id(0) * RPT + lax.broadcasted_iota(jnp.int32, (RPT, 1), 0)
    y2_ref[...] = jnp.where(row < N, y2, 0.0)


def _tc3_body(a_ref, y2_ref, d_ref, b_ref, o_ref):
    o_ref[...] = d_ref[...] * (a_ref[0] + a_ref[1] + y2_ref[...]) + b_ref[...]


def _row_spec(shape_last):
    return pl.BlockSpec((RPT, shape_last), lambda i: (i, 0))


_AGG_SPEC = pl.BlockSpec((NC, RPT, D), lambda i: (0, i, 0))
_FULL_W = pl.BlockSpec((D, D), lambda i: (0, 0))
_FULL_B = pl.BlockSpec((1, D), lambda i: (0, 0))

_tc1 = pl.pallas_call(
    _tc1_body,
    grid=(_GRID,),
    in_specs=[_row_spec(1), _row_spec(1), _row_spec(D), _FULL_W],
    out_specs=[_row_spec(D), _row_spec(1)],
    out_shape=[jax.ShapeDtypeStruct((NPAD, D), jnp.float32),
               jax.ShapeDtypeStruct((NPAD, 1), jnp.float32)],
)

_tc2 = pl.pallas_call(
    _tc2_body,
    grid=(_GRID,),
    in_specs=[_AGG_SPEC, _row_spec(D), _row_spec(1), _FULL_B, _FULL_W],
    out_specs=_row_spec(D),
    out_shape=jax.ShapeDtypeStruct((NPAD, D), jnp.float32),
)

_R3 = 1000
_tc3 = pl.pallas_call(
    _tc3_body,
    grid=(N // _R3,),
    in_specs=[pl.BlockSpec((NC, _R3, D), lambda i: (0, i, 0)),
              pl.BlockSpec((_R3, D), lambda i: (i, 0)),
              pl.BlockSpec((_R3, 1), lambda i: (i, 0)),
              _FULL_B],
    out_specs=pl.BlockSpec((_R3, D), lambda i: (i, 0)),
    out_shape=jax.ShapeDtypeStruct((N, D), jnp.float32),
)


@jax.jit
def kernel(x, edge_index, W1, b1, W2, b2):
    src = edge_index[0].astype(jnp.int32)
    dst = edge_index[1].astype(jnp.int32)
    pad = jnp.arange(EPAD - E, dtype=jnp.int32)
    src_p = jnp.concatenate(
        [src, N + pad % (NPAD - N)]).reshape(NW, EPT)
    dst_p = jnp.concatenate(
        [dst, pad % N]).reshape(NW, NPHASE, PCH, K)
    dst_deg = jnp.concatenate(
        [dst, jnp.full((EPAD - E,), TRASH, jnp.int32)]).reshape(NW, EPT)

    deg_p = _deg_kernel(dst_deg)
    dega = deg_p[0].reshape(NPAD, 1)
    degb = deg_p[1].reshape(NPAD, 1)

    x_pad = jnp.pad(x, ((0, NPAD - N), (0, 0)))
    y1, dcol = _tc1(dega, degb, x_pad, W1)

    agg1 = _agg_kernel(y1, src_p, dst_p)
    y2 = _tc2(agg1, y1, dcol, b1.reshape(1, D), W2)

    agg2 = _agg_kernel(y2, src_p, dst_p)
    return _tc3(agg2, y2, dcol, b2.reshape(1, D))


# speedup vs baseline: 2.8510x; 1.0178x over previous
import jax
import jax.numpy as jnp
from jax import lax
from jax.experimental import pallas as pl
from jax.experimental.pallas import tpu as pltpu
from jax.experimental.pallas import tpu_sc as plsc

N = 10000
D = 128
E = 320000

NC = 2
NS = 16
NW = NC * NS
K = 128
EPAD = 327680
EPT = EPAD // NW
NPHASE = 2
PCH = EPT // (NPHASE * K)
PE = PCH * K

NPAD = 10240
RPT = NPAD // NS
TRASH = N

_MESH = plsc.VectorSubcoreMesh(core_axis_name="c", subcore_axis_name="s")
_SC_PARAMS = pltpu.CompilerParams(needs_layout_passes=False)


def _deg_body(dst_hbm, deg_out, dst_v, acc_v, tbuf, rbuf, deg_sh, sem):
    c = lax.axis_index("c")
    s = lax.axis_index("s")
    wid = c * NS + s

    pltpu.async_copy(dst_hbm.at[wid], dst_v, sem).wait()

    z16 = jnp.zeros((16,), jnp.float32)

    @pl.loop(0, NPAD, step=16)
    def _(i):
        acc_v[pl.ds(i, 16)] = z16

    ones16 = jnp.ones((16,), jnp.float32)

    @pl.loop(0, EPT, step=16)
    def _(e):
        idx = dst_v[pl.ds(e, 16)]
        plsc.addupdate_scatter(acc_v, [idx], ones16)

    pltpu.sync_copy(acc_v, deg_sh.at[s])
    plsc.subcore_barrier()

    @pl.loop(0, RPT, step=16)
    def _(i):
        rbuf[pl.ds(i, 16)] = z16

    for j in range(NS):
        pltpu.sync_copy(deg_sh.at[j, pl.ds(s * RPT, RPT)], tbuf)

        @pl.loop(0, RPT, step=16)
        def _(i):
            rbuf[pl.ds(i, 16)] = rbuf[pl.ds(i, 16)] + tbuf[pl.ds(i, 16)]

    pltpu.sync_copy(rbuf, deg_out.at[c, pl.ds(s * RPT, RPT)])


_deg_kernel = pl.kernel(
    _deg_body,
    out_type=jax.ShapeDtypeStruct((NC, NPAD), jnp.float32),
    mesh=_MESH,
    compiler_params=_SC_PARAMS,
    scratch_types=[
        pltpu.VMEM((EPT,), jnp.int32),
        pltpu.VMEM((NPAD,), jnp.float32),
        pltpu.VMEM((RPT,), jnp.float32),
        pltpu.VMEM((RPT,), jnp.float32),
        pltpu.VMEM_SHARED((NS, NPAD), jnp.float32),
        pltpu.SemaphoreType.DMA,
    ],
)


def _agg_body(y_hbm, src_hbm, dst_hbm, out_hbm,
              src_v, dst_v, rbuf0, rbuf1, acc_sh, sem0, sem1, sem2, sem3):
    c = lax.axis_index("c")
    s = lax.axis_index("s")
    wid = c * NS + s

    z16 = jnp.zeros((16,), jnp.float32)

    @pl.loop(0, K)
    def _(r):
        for i in range(D // 16):
            rbuf0[r, pl.ds(i * 16, 16)] = z16

    for t in range(RPT // K):
        pltpu.sync_copy(rbuf0, acc_sh.at[pl.ds(s * RPT + t * K, K)])
    plsc.subcore_barrier()

    H = K // 2

    def gather(j, rbuf, sem):
        pltpu.async_copy(y_hbm.at[src_v.at[pl.ds(j * K, H)]],
                         rbuf.at[pl.ds(0, H)], sem)
        pltpu.async_copy(y_hbm.at[src_v.at[pl.ds(j * K + H, H)]],
                         rbuf.at[pl.ds(H, H)], sem)

    def gather_wait(j, rbuf, sem):
        pltpu.make_async_copy(y_hbm.at[src_v.at[pl.ds(j * K, H)]],
                              rbuf.at[pl.ds(0, H)], sem).wait()
        pltpu.make_async_copy(y_hbm.at[src_v.at[pl.ds(j * K + H, H)]],
                              rbuf.at[pl.ds(H, H)], sem).wait()

    def scatter_fire(j, rbuf, sem):
        pltpu.async_copy(rbuf, acc_sh.at[dst_v.at[j]], sem, add=True)

    def scatter_drain(j, rbuf, sem):
        pltpu.make_async_copy(rbuf, acc_sh.at[dst_v.at[j]], sem).wait()

    for p in range(NPHASE):
        pltpu.async_copy(src_hbm.at[wid, pl.ds(p * PE, PE)], src_v,
                         sem0).wait()
        pltpu.async_copy(dst_hbm.at[wid, p], dst_v, sem1).wait()
        gather(0, rbuf0, sem0)
        gather(1, rbuf1, sem1)

        @pl.loop(0, PCH, step=2)
        def _(j):
            gather_wait(j, rbuf0, sem0)
            scatter_fire(j, rbuf0, sem2)
            gather_wait(j + 1, rbuf1, sem1)
            scatter_fire(j + 1, rbuf1, sem3)

            @pl.when(j + 2 < PCH)
            def _():
                scatter_drain(j, rbuf0, sem2)
                gather(j + 2, rbuf0, sem0)
                scatter_drain(j + 1, rbuf1, sem3)
                gather(j + 3, rbuf1, sem1)

        scatter_drain(PCH - 2, rbuf0, sem2)
        scatter_drain(PCH - 1, rbuf1, sem3)

    plsc.subcore_barrier()

    bufs = (rbuf0, rbuf1)
    sems = (sem0, sem1)

    def orow(t):
        return out_hbm.at[c, pl.ds(s * RPT + t * K, K)]

    for t in range(RPT // K):
        if t >= 2:
            pltpu.make_async_copy(bufs[(t - 2) % 2], orow(t - 2),
                                  sems[(t - 2) % 2]).wait()
        pltpu.sync_copy(acc_sh.at[pl.ds(s * RPT + t * K, K)], bufs[t % 2])
        pltpu.async_copy(bufs[t % 2], orow(t), sems[t % 2])
    for t in (RPT // K - 2, RPT // K - 1):
        pltpu.make_async_copy(bufs[t % 2], orow(t), sems[t % 2]).wait()


_agg_kernel = pl.kernel(
    _agg_body,
    out_type=jax.ShapeDtypeStruct((NC, NPAD, D), jnp.float32),
    mesh=_MESH,
    compiler_params=_SC_PARAMS,
    scratch_types=[
        pltpu.VMEM((PE,), jnp.int32),
        pltpu.VMEM((PCH, K), jnp.int32),
        pltpu.VMEM((K, D), jnp.float32),
        pltpu.VMEM((K, D), jnp.float32),
        pltpu.VMEM_SHARED((NPAD, D), jnp.float32),
        pltpu.SemaphoreType.DMA,
        pltpu.SemaphoreType.DMA,
        pltpu.SemaphoreType.DMA,
        pltpu.SemaphoreType.DMA,
    ],
)


_GRID = NPAD // RPT


def _tc0_body(x_ref, w_ref, xw_ref):
    xw_ref[...] = jnp.dot(x_ref[...], w_ref[...],
                          preferred_element_type=jnp.float32,
                          precision=lax.Precision.HIGHEST)


def _tc1_body(dega_ref, degb_ref, xw_ref, y_ref, d_ref):
    d = lax.rsqrt(1.0 + dega_ref[...] + degb_ref[...])
    y_ref[...] = xw_ref[...] * d
    d_ref[...] = d


def _tc2_body(a_ref, y1_ref, d_ref, b_ref, w_ref, y2_ref):
    d = d_ref[...]
    h = d * (a_ref[0] + a_ref[1] + y1_ref[...]) + b_ref[...]
    h = jnp.maximum(h, 0.0)
    y2 = d * jnp.dot(h, w_ref[...],
                     preferred_element_type=jnp.float32,
                     precision=lax.Precision.HIGHEST)
    row = pl.program_id(0) * RPT + lax.broadcasted_iota(jnp.int32, (RPT, 1), 0)
    y2_ref[...] = jnp.where(row < N, y2, 0.0)


def _tc3_body(a_ref, y2_ref, d_ref, b_ref, o_ref):
    o_ref[...] = d_ref[...] * (a_ref[0] + a_ref[1] + y2_ref[...]) + b_ref[...]


def _row_spec(shape_last):
    return pl.BlockSpec((RPT, shape_last), lambda i: (i, 0))


_AGG_SPEC = pl.BlockSpec((NC, RPT, D), lambda i: (0, i, 0))
_FULL_W = pl.BlockSpec((D, D), lambda i: (0, 0))
_FULL_B = pl.BlockSpec((1, D), lambda i: (0, 0))

_tc0 = pl.pallas_call(
    _tc0_body,
    grid=(_GRID,),
    in_specs=[_row_spec(D), _FULL_W],
    out_specs=_row_spec(D),
    out_shape=jax.ShapeDtypeStruct((NPAD, D), jnp.float32),
)

_tc1 = pl.pallas_call(
    _tc1_body,
    grid=(_GRID,),
    in_specs=[_row_spec(1), _row_spec(1), _row_spec(D)],
    out_specs=[_row_spec(D), _row_spec(1)],
    out_shape=[jax.ShapeDtypeStruct((NPAD, D), jnp.float32),
               jax.ShapeDtypeStruct((NPAD, 1), jnp.float32)],
)

_tc2 = pl.pallas_call(
    _tc2_body,
    grid=(_GRID,),
    in_specs=[_AGG_SPEC, _row_spec(D), _row_spec(1), _FULL_B, _FULL_W],
    out_specs=_row_spec(D),
    out_shape=jax.ShapeDtypeStruct((NPAD, D), jnp.float32),
)

_R3 = 1000
_tc3 = pl.pallas_call(
    _tc3_body,
    grid=(N // _R3,),
    in_specs=[pl.BlockSpec((NC, _R3, D), lambda i: (0, i, 0)),
              pl.BlockSpec((_R3, D), lambda i: (i, 0)),
              pl.BlockSpec((_R3, 1), lambda i: (i, 0)),
              _FULL_B],
    out_specs=pl.BlockSpec((_R3, D), lambda i: (i, 0)),
    out_shape=jax.ShapeDtypeStruct((N, D), jnp.float32),
)


@jax.jit
def kernel(x, edge_index, W1, b1, W2, b2):
    src = edge_index[0].astype(jnp.int32)
    dst = edge_index[1].astype(jnp.int32)
    pad = jnp.arange(EPAD - E, dtype=jnp.int32)
    src_p = jnp.concatenate(
        [src, N + pad % (NPAD - N)]).reshape(NW, EPT)
    dst_p = jnp.concatenate(
        [dst, pad % N]).reshape(NW, NPHASE, PCH, K)
    dst_deg = jnp.concatenate(
        [dst, jnp.full((EPAD - E,), TRASH, jnp.int32)]).reshape(NW, EPT)

    x_pad = jnp.pad(x, ((0, NPAD - N), (0, 0)))
    xw = _tc0(x_pad, W1)
    deg_p = _deg_kernel(dst_deg)
    dega = deg_p[0].reshape(NPAD, 1)
    degb = deg_p[1].reshape(NPAD, 1)
    y1, dcol = _tc1(dega, degb, xw)

    agg1 = _agg_kernel(y1, src_p, dst_p)
    y2 = _tc2(agg1, y1, dcol, b1.reshape(1, D), W2)

    agg2 = _agg_kernel(y2, src_p, dst_p)
    return _tc3(agg2, y2, dcol, b2.reshape(1, D))


# speedup vs baseline: 2.9010x; 1.0176x over previous
import jax
import jax.numpy as jnp
from jax import lax
from jax.experimental import pallas as pl
from jax.experimental.pallas import tpu as pltpu
from jax.experimental.pallas import tpu_sc as plsc

N = 10000
D = 128
E = 320000

NC = 2
NS = 16
NW = NC * NS
K = 128
EPAD = 327680
EPT = EPAD // NW
NPHASE = 2
PCH = EPT // (NPHASE * K)
PE = PCH * K

NPAD = 10240
RPT = NPAD // NS

_MESH = plsc.VectorSubcoreMesh(core_axis_name="c", subcore_axis_name="s")
_SC_PARAMS = pltpu.CompilerParams(needs_layout_passes=False)


def _deg_body(dst_hbm, deg_out, dst_v, acc_v, tbuf, rbuf, deg_sh, sem):
    c = lax.axis_index("c")
    s = lax.axis_index("s")
    wid = c * NS + s

    pltpu.async_copy(dst_hbm.at[wid], dst_v, sem).wait()

    z16 = jnp.zeros((16,), jnp.float32)

    @pl.loop(0, NPAD, step=16)
    def _(i):
        acc_v[pl.ds(i, 16)] = z16

    ones16 = jnp.ones((16,), jnp.float32)

    @pl.loop(0, EPT, step=16)
    def _(e):
        idx = dst_v[pl.ds(e, 16)]
        plsc.addupdate_scatter(acc_v, [idx], ones16)

    pltpu.sync_copy(acc_v, deg_sh.at[s])
    plsc.subcore_barrier()

    @pl.loop(0, RPT, step=16)
    def _(i):
        rbuf[pl.ds(i, 16)] = z16

    for j in range(NS):
        pltpu.sync_copy(deg_sh.at[j, pl.ds(s * RPT, RPT)], tbuf)

        @pl.loop(0, RPT, step=16)
        def _(i):
            rbuf[pl.ds(i, 16)] = rbuf[pl.ds(i, 16)] + tbuf[pl.ds(i, 16)]

    pltpu.sync_copy(rbuf, deg_out.at[c, pl.ds(s * RPT, RPT)])


_deg_kernel = pl.kernel(
    _deg_body,
    out_type=jax.ShapeDtypeStruct((NC, NPAD), jnp.float32),
    mesh=_MESH,
    compiler_params=_SC_PARAMS,
    scratch_types=[
        pltpu.VMEM((EPT,), jnp.int32),
        pltpu.VMEM((NPAD,), jnp.float32),
        pltpu.VMEM((RPT,), jnp.float32),
        pltpu.VMEM((RPT,), jnp.float32),
        pltpu.VMEM_SHARED((NS, NPAD), jnp.float32),
        pltpu.SemaphoreType.DMA,
    ],
)


def _agg_body(y_hbm, src_hbm, dst_hbm, out_hbm,
              src_v, dst_v, rbuf0, rbuf1, acc_sh, sem0, sem1, sem2, sem3):
    c = lax.axis_index("c")
    s = lax.axis_index("s")
    wid = c * NS + s

    z16 = jnp.zeros((16,), jnp.float32)

    @pl.loop(0, K)
    def _(r):
        for i in range(D // 16):
            rbuf0[r, pl.ds(i * 16, 16)] = z16

    for t in range(RPT // K):
        pltpu.sync_copy(rbuf0, acc_sh.at[pl.ds(s * RPT + t * K, K)])
    plsc.subcore_barrier()

    H = K // 2

    def gather(j, rbuf, sem):
        pltpu.async_copy(y_hbm.at[src_v.at[pl.ds(j * K, H)]],
                         rbuf.at[pl.ds(0, H)], sem)
        pltpu.async_copy(y_hbm.at[src_v.at[pl.ds(j * K + H, H)]],
                         rbuf.at[pl.ds(H, H)], sem)

    def gather_wait(j, rbuf, sem):
        pltpu.make_async_copy(y_hbm.at[src_v.at[pl.ds(j * K, H)]],
                              rbuf.at[pl.ds(0, H)], sem).wait()
        pltpu.make_async_copy(y_hbm.at[src_v.at[pl.ds(j * K + H, H)]],
                              rbuf.at[pl.ds(H, H)], sem).wait()

    def scatter_fire(j, rbuf, sem):
        pltpu.async_copy(rbuf, acc_sh.at[dst_v.at[j]], sem, add=True)

    def scatter_drain(j, rbuf, sem):
        pltpu.make_async_copy(rbuf, acc_sh.at[dst_v.at[j]], sem).wait()

    for p in range(NPHASE):
        pltpu.async_copy(src_hbm.at[wid, pl.ds(p * PE, PE)], src_v,
                         sem0).wait()
        pltpu.async_copy(dst_hbm.at[wid, p], dst_v, sem1).wait()
        gather(0, rbuf0, sem0)
        gather(1, rbuf1, sem1)

        @pl.loop(0, PCH, step=2)
        def _(j):
            gather_wait(j, rbuf0, sem0)
            scatter_fire(j, rbuf0, sem2)
            gather_wait(j + 1, rbuf1, sem1)
            scatter_fire(j + 1, rbuf1, sem3)

            @pl.when(j + 2 < PCH)
            def _():
                scatter_drain(j, rbuf0, sem2)
                gather(j + 2, rbuf0, sem0)
                scatter_drain(j + 1, rbuf1, sem3)
                gather(j + 3, rbuf1, sem1)

        scatter_drain(PCH - 2, rbuf0, sem2)
        scatter_drain(PCH - 1, rbuf1, sem3)

    plsc.subcore_barrier()

    bufs = (rbuf0, rbuf1)
    sems = (sem0, sem1)

    def orow(t):
        return out_hbm.at[c, pl.ds(s * RPT + t * K, K)]

    for t in range(RPT // K):
        if t >= 2:
            pltpu.make_async_copy(bufs[(t - 2) % 2], orow(t - 2),
                                  sems[(t - 2) % 2]).wait()
        pltpu.sync_copy(acc_sh.at[pl.ds(s * RPT + t * K, K)], bufs[t % 2])
        pltpu.async_copy(bufs[t % 2], orow(t), sems[t % 2])
    for t in (RPT // K - 2, RPT // K - 1):
        pltpu.make_async_copy(bufs[t % 2], orow(t), sems[t % 2]).wait()


_agg_kernel = pl.kernel(
    _agg_body,
    out_type=jax.ShapeDtypeStruct((NC, NPAD, D), jnp.float32),
    mesh=_MESH,
    compiler_params=_SC_PARAMS,
    scratch_types=[
        pltpu.VMEM((PE,), jnp.int32),
        pltpu.VMEM((PCH, K), jnp.int32),
        pltpu.VMEM((K, D), jnp.float32),
        pltpu.VMEM((K, D), jnp.float32),
        pltpu.VMEM_SHARED((NPAD, D), jnp.float32),
        pltpu.SemaphoreType.DMA,
        pltpu.SemaphoreType.DMA,
        pltpu.SemaphoreType.DMA,
        pltpu.SemaphoreType.DMA,
    ],
)


_GRID = NPAD // RPT


NSPUR = EPAD - E


def _tc0_body(x_ref, w_ref, xw_ref):
    xw_ref[...] = jnp.dot(x_ref[...], w_ref[...],
                          preferred_element_type=jnp.float32,
                          precision=lax.Precision.HIGHEST)


def _tc1_body(dega_ref, degb_ref, xw_ref, y_ref, d_ref):
    row = pl.program_id(0) * RPT + lax.broadcasted_iota(jnp.int32, (RPT, 1), 0)
    spur = jnp.where(row < NSPUR, 1.0, 0.0)
    d = lax.rsqrt(1.0 + dega_ref[...] + degb_ref[...] - spur)
    y_ref[...] = jnp.where(row < N, xw_ref[...] * d, 0.0)
    d_ref[...] = d


def _tc2_body(a_ref, y1_ref, d_ref, b_ref, w_ref, y2_ref):
    d = d_ref[...]
    h = d * (a_ref[0] + a_ref[1] + y1_ref[...]) + b_ref[...]
    h = jnp.maximum(h, 0.0)
    y2 = d * jnp.dot(h, w_ref[...],
                     preferred_element_type=jnp.float32,
                     precision=lax.Precision.HIGHEST)
    row = pl.program_id(0) * RPT + lax.broadcasted_iota(jnp.int32, (RPT, 1), 0)
    y2_ref[...] = jnp.where(row < N, y2, 0.0)


def _tc3_body(a_ref, y2_ref, d_ref, b_ref, o_ref):
    o_ref[...] = d_ref[...] * (a_ref[0] + a_ref[1] + y2_ref[...]) + b_ref[...]


def _row_spec(shape_last):
    return pl.BlockSpec((RPT, shape_last), lambda i: (i, 0))


_AGG_SPEC = pl.BlockSpec((NC, RPT, D), lambda i: (0, i, 0))
_FULL_W = pl.BlockSpec((D, D), lambda i: (0, 0))
_FULL_B = pl.BlockSpec((1, D), lambda i: (0, 0))

_tc0 = pl.pallas_call(
    _tc0_body,
    grid=(_GRID,),
    in_specs=[pl.BlockSpec((RPT, D), lambda i: (i, 0)), _FULL_W],
    out_specs=_row_spec(D),
    out_shape=jax.ShapeDtypeStruct((NPAD, D), jnp.float32),
)

_tc1 = pl.pallas_call(
    _tc1_body,
    grid=(_GRID,),
    in_specs=[_row_spec(1), _row_spec(1), _row_spec(D)],
    out_specs=[_row_spec(D), _row_spec(1)],
    out_shape=[jax.ShapeDtypeStruct((NPAD, D), jnp.float32),
               jax.ShapeDtypeStruct((NPAD, 1), jnp.float32)],
)

_tc2 = pl.pallas_call(
    _tc2_body,
    grid=(_GRID,),
    in_specs=[_AGG_SPEC, _row_spec(D), _row_spec(1), _FULL_B, _FULL_W],
    out_specs=_row_spec(D),
    out_shape=jax.ShapeDtypeStruct((NPAD, D), jnp.float32),
)

_R3 = 1000
_tc3 = pl.pallas_call(
    _tc3_body,
    grid=(N // _R3,),
    in_specs=[pl.BlockSpec((NC, _R3, D), lambda i: (0, i, 0)),
              pl.BlockSpec((_R3, D), lambda i: (i, 0)),
              pl.BlockSpec((_R3, 1), lambda i: (i, 0)),
              _FULL_B],
    out_specs=pl.BlockSpec((_R3, D), lambda i: (i, 0)),
    out_shape=jax.ShapeDtypeStruct((N, D), jnp.float32),
)


@jax.jit
def kernel(x, edge_index, W1, b1, W2, b2):
    src = edge_index[0].astype(jnp.int32)
    dst = edge_index[1].astype(jnp.int32)
    pad = jnp.arange(EPAD - E, dtype=jnp.int32)
    src_p = jnp.concatenate(
        [src, N + pad % (NPAD - N)]).reshape(NW, EPT)
    dst_p = jnp.concatenate([dst, pad]).reshape(NW, NPHASE, PCH, K)

    xw = _tc0(x, W1)
    deg_p = _deg_kernel(dst_p.reshape(NW, EPT))
    dega = deg_p[0].reshape(NPAD, 1)
    degb = deg_p[1].reshape(NPAD, 1)
    y1, dcol = _tc1(dega, degb, xw)

    agg1 = _agg_kernel(y1, src_p, dst_p)
    y2 = _tc2(agg1, y1, dcol, b1.reshape(1, D), W2)

    agg2 = _agg_kernel(y2, src_p, dst_p)
    return _tc3(agg2, y2, dcol, b2.reshape(1, D))
